# baseline (speedup 1.0000x reference)
import functools

import jax
import jax.numpy as jnp
import numpy as np
from jax.experimental import pallas as pl
from jax.experimental.pallas import tpu as pltpu

HID = 1024
NH = 16
NKV = 4
HD = 64
QM = NH // NKV
NE = 16
TOPK = 2
INTER = 1024
LIMIT = 7.0
THETA = 150000.0
CACHE = 4096
B = 8

INTERPRET = False


def _rope_cos_sin(pos):
    half = HD // 2
    i = jax.lax.broadcasted_iota(jnp.int32, (1, half), 1).astype(jnp.float32)
    inv_freq = jnp.exp(-(i / half) * np.log(THETA))
    freqs = pos * inv_freq
    return jnp.cos(freqs), jnp.sin(freqs)


def _qkv_kernel(x_ref, scale_ref, w_ref, b_ref, q_ref, k_ref, v_ref):
    x = x_ref[...]
    xs = x * jax.lax.rsqrt(jnp.mean(x * x, axis=-1, keepdims=True) + 1e-5)
    t = xs * scale_ref[...]
    qkv = jax.lax.dot_general(t, w_ref[...], (((1,), (1,)), ((), ())),
                              preferred_element_type=jnp.float32) + b_ref[...]
    q_ref[...] = qkv[:, : NH * HD]
    k = qkv[:, NH * HD:(NH + NKV) * HD]
    v = qkv[:, (NH + NKV) * HD:]
    cos, sin = _rope_cos_sin(float(CACHE))
    half = HD // 2
    pieces = []
    for h in range(NKV):
        x1 = k[:, h * HD: h * HD + half]
        x2 = k[:, h * HD + half: (h + 1) * HD]
        pieces.append(x1 * cos - x2 * sin)
        pieces.append(x2 * cos + x1 * sin)
    k_ref[...] = jnp.concatenate(pieces, axis=1)
    v_ref[...] = v


def _attn_kernel(q_ref, ck_ref, cv_ref, kn_ref, vn_ref, sink_ref,
                 attn_ref, ko_ref, vo_ref):
    sm_scale = 1.0 / np.sqrt(HD)
    half = HD // 2
    q16 = q_ref[...]
    cos, sin = _rope_cos_sin(float(CACHE))
    q1 = q16[:, :half]
    q2 = q16[:, half:]
    q16 = jnp.concatenate([q1 * cos - q2 * sin, q2 * cos + q1 * sin], axis=1)
    row = jax.lax.broadcasted_iota(jnp.int32, (NH, 1), 0) // QM
    q_exp = jnp.concatenate(
        [jnp.where(row == g, q16, 0.0) for g in range(NKV)], axis=1)

    K2 = ck_ref[0]
    V2 = cv_ref[0]
    kn = kn_ref[0]
    vn = vn_ref[0]

    scores = jax.lax.dot_general(q_exp, K2, (((1,), (1,)), ((), ())),
                                 preferred_element_type=jnp.float32) * sm_scale
    s_new = jax.lax.dot_general(q_exp, kn, (((1,), (1,)), ((), ())),
                                preferred_element_type=jnp.float32) * sm_scale
    s_sink = sink_ref[...]
    m = jnp.maximum(jnp.max(scores, axis=1, keepdims=True),
                    jnp.maximum(s_new, s_sink))
    p = jnp.exp(scores - m)
    p_new = jnp.exp(s_new - m)
    denom = (jnp.sum(p, axis=1, keepdims=True) + p_new
             + jnp.exp(s_sink - m))
    attn_all = jax.lax.dot_general(p, V2, (((1,), (0,)), ((), ())),
                                   preferred_element_type=jnp.float32)
    attn_all = (attn_all + p_new * vn) / denom
    attn_ref[...] = jnp.concatenate(
        [attn_all[g * QM:(g + 1) * QM, g * HD:(g + 1) * HD]
         for g in range(NKV)], axis=0)
    ko_ref[0, pl.ds(0, CACHE), :] = K2
    vo_ref[0, pl.ds(0, CACHE), :] = V2
    ko_ref[0, pl.ds(CACHE, 1), :] = kn
    vo_ref[0, pl.ds(CACHE, 1), :] = vn


def _post_kernel(x_ref, attn_ref, ow_ref, ob_ref, ms_ref, gw_ref, gb_ref,
                 x1_ref, t2_ref, c_ref, sched_ref):
    x1 = x_ref[...] + jax.lax.dot_general(
        attn_ref[...], ow_ref[...], (((1,), (1,)), ((), ())),
        preferred_element_type=jnp.float32) + ob_ref[...]
    x1_ref[...] = x1
    xs = x1 * jax.lax.rsqrt(jnp.mean(x1 * x1, axis=-1, keepdims=True) + 1e-5)
    t2 = xs * ms_ref[...]
    t2_ref[...] = t2
    g = jax.lax.dot_general(t2, gw_ref[...], (((1,), (1,)), ((), ())),
                            preferred_element_type=jnp.float32) + gb_ref[...]
    iota = jax.lax.broadcasted_iota(jnp.int32, (B, NE), 1)
    m1 = jnp.max(g, axis=1, keepdims=True)
    idx1 = jnp.min(jnp.where(g == m1, iota, NE), axis=1, keepdims=True)
    g2 = jnp.where(iota == idx1, -jnp.inf, g)
    m2 = jnp.max(g2, axis=1, keepdims=True)
    idx2 = jnp.min(jnp.where(g2 == m2, iota, NE), axis=1, keepdims=True)
    p1 = 1.0 / (1.0 + jnp.exp(m2 - m1))
    p2 = 1.0 - p1
    C = (jnp.where(iota == idx1, p1, 0.0)
         + jnp.where(iota == idx2, p2, 0.0))
    c_ref[...] = C
    usedf = jnp.max(jnp.where(C != 0.0, 1.0, 0.0), axis=0, keepdims=True)
    le = jax.lax.broadcasted_iota(jnp.int32, (NE, NE), 0)
    ge = jax.lax.broadcasted_iota(jnp.int32, (NE, NE), 1)
    lower = jnp.where(le <= ge, 1.0, 0.0)
    rank = jax.lax.dot_general(usedf, lower, (((1,), (0,)), ((), ())),
                               preferred_element_type=jnp.float32)
    lane = jax.lax.broadcasted_iota(jnp.int32, (NE, NE), 1)
    slot = jax.lax.broadcasted_iota(jnp.int32, (NE, NE), 0)
    cond = (usedf > 0.0) & (rank.astype(jnp.int32) == slot + 1)
    sched = jnp.max(jnp.where(cond, lane, -1), axis=1, keepdims=True)
    maxused = jnp.max(jnp.where(usedf > 0.0, lane[:1], -1), axis=1, keepdims=True)
    sched_ref[...] = jnp.where(sched < 0, maxused, sched)


def _moe_kernel(s_ref, t2_ref, x1_ref, c_ref, w1_ref, b1g_ref, b1l_ref,
                w2_ref, b2_ref, out_ref):
    i = pl.program_id(0)
    e = s_ref[i]
    t2 = t2_ref[...]
    w1 = w1_ref[0]
    w1r = jnp.reshape(w1, (INTER, 2, HID))
    hg = jax.lax.dot_general(t2, w1r[:, 0, :], (((1,), (1,)), ((), ())),
                             preferred_element_type=jnp.float32) + b1g_ref[0]
    hl = jax.lax.dot_general(t2, w1r[:, 1, :], (((1,), (1,)), ((), ())),
                             preferred_element_type=jnp.float32) + b1l_ref[0]
    xg = jnp.minimum(hg, LIMIT)
    xl = jnp.clip(hl, -LIMIT, LIMIT)
    act = xg * jax.nn.sigmoid(1.702 * xg) * (xl + 1.0)
    o = jax.lax.dot_general(act, w2_ref[0], (((1,), (1,)), ((), ())),
                            preferred_element_type=jnp.float32) + b2_ref[0]
    lane = jax.lax.broadcasted_iota(jnp.int32, (B, NE), 1)
    C = c_ref[...]
    ce = jnp.sum(jnp.where(lane == e, C, 0.0), axis=1, keepdims=True)
    n_used = jnp.sum(jnp.max(jnp.where(C != 0.0, 1.0, 0.0), axis=0))
    ce = jnp.where(i.astype(jnp.float32) < n_used, ce, 0.0)

    @pl.when(i == 0)
    def _():
        out_ref[...] = x1_ref[...]

    out_ref[...] += ce * o


@jax.jit
def kernel(x, cache_k, cache_v, sinks, attn_norm_scale, qkv_w, qkv_b,
           out_w, out_b, mlp_norm_scale, gate_w, gate_b,
           mlp1_w, mlp1_b, mlp2_w, mlp2_b):
    Bq, Tq, _ = x.shape
    qkv_dim = HD * (NH + 2 * NKV)
    x2 = x.reshape(Bq, HID)

    q, k_new, v_new = pl.pallas_call(
        _qkv_kernel,
        out_shape=[
            jax.ShapeDtypeStruct((Bq, NH * HD), jnp.float32),
            jax.ShapeDtypeStruct((Bq, NKV * HD), jnp.float32),
            jax.ShapeDtypeStruct((Bq, NKV * HD), jnp.float32),
        ],
        interpret=INTERPRET,
    )(x2, attn_norm_scale.reshape(1, HID), qkv_w, qkv_b.reshape(1, qkv_dim))

    q128 = q.reshape(Bq * NH, HD)
    ck = cache_k.reshape(Bq, CACHE, NKV * HD)
    cv = cache_v.reshape(Bq, CACHE, NKV * HD)

    attn, K_out, V_out = pl.pallas_call(
        _attn_kernel,
        grid=(Bq,),
        in_specs=[
            pl.BlockSpec((NH, HD), lambda b: (b, 0)),
            pl.BlockSpec((1, CACHE, NKV * HD), lambda b: (b, 0, 0)),
            pl.BlockSpec((1, CACHE, NKV * HD), lambda b: (b, 0, 0)),
            pl.BlockSpec((1, 1, NKV * HD), lambda b: (b, 0, 0)),
            pl.BlockSpec((1, 1, NKV * HD), lambda b: (b, 0, 0)),
            pl.BlockSpec((NH, 1), lambda b: (0, 0)),
        ],
        out_specs=[
            pl.BlockSpec((NH, HD), lambda b: (b, 0)),
            pl.BlockSpec((1, CACHE + 1, NKV * HD), lambda b: (b, 0, 0)),
            pl.BlockSpec((1, CACHE + 1, NKV * HD), lambda b: (b, 0, 0)),
        ],
        out_shape=[
            jax.ShapeDtypeStruct((Bq * NH, HD), jnp.float32),
            jax.ShapeDtypeStruct((Bq, CACHE + 1, NKV * HD), jnp.float32),
            jax.ShapeDtypeStruct((Bq, CACHE + 1, NKV * HD), jnp.float32),
        ],
        interpret=INTERPRET,
    )(q128, ck, cv, k_new.reshape(Bq, 1, NKV * HD), v_new.reshape(Bq, 1, NKV * HD),
      sinks.reshape(NH, 1))

    x1, t2, C, sched = pl.pallas_call(
        _post_kernel,
        out_shape=[
            jax.ShapeDtypeStruct((Bq, HID), jnp.float32),
            jax.ShapeDtypeStruct((Bq, HID), jnp.float32),
            jax.ShapeDtypeStruct((Bq, NE), jnp.float32),
            jax.ShapeDtypeStruct((NE, 1), jnp.int32),
        ],
        interpret=INTERPRET,
    )(x2, attn.reshape(Bq, NH * HD), out_w, out_b.reshape(1, HID),
      mlp_norm_scale.reshape(1, HID), gate_w, gate_b.reshape(1, NE))

    b1r = mlp1_b.reshape(NE, INTER, 2)
    out = pl.pallas_call(
        _moe_kernel,
        grid_spec=pltpu.PrefetchScalarGridSpec(
            num_scalar_prefetch=1,
            grid=(NE,),
            in_specs=[
                pl.BlockSpec((Bq, HID), lambda i, s: (0, 0)),
                pl.BlockSpec((Bq, HID), lambda i, s: (0, 0)),
                pl.BlockSpec((Bq, NE), lambda i, s: (0, 0)),
                pl.BlockSpec((1, 2 * INTER, HID), lambda i, s: (s[i], 0, 0)),
                pl.BlockSpec((1, 1, INTER), lambda i, s: (s[i], 0, 0)),
                pl.BlockSpec((1, 1, INTER), lambda i, s: (s[i], 0, 0)),
                pl.BlockSpec((1, HID, INTER), lambda i, s: (s[i], 0, 0)),
                pl.BlockSpec((1, 1, HID), lambda i, s: (s[i], 0, 0)),
            ],
            out_specs=pl.BlockSpec((Bq, HID), lambda i, s: (0, 0)),
        ),
        out_shape=jax.ShapeDtypeStruct((Bq, HID), jnp.float32),
        interpret=INTERPRET,
    )(sched.reshape(NE), t2, x1, C, mlp1_w, b1r[:, :, 0].reshape(NE, 1, INTER),
      b1r[:, :, 1].reshape(NE, 1, INTER), mlp2_w, mlp2_b.reshape(NE, 1, HID))

    return (out.reshape(Bq, Tq, HID),
            K_out.reshape(Bq, CACHE + 1, NKV, HD),
            V_out.reshape(Bq, CACHE + 1, NKV, HD))


# speedup vs baseline: 1.0718x; 1.0718x over previous
import functools

import jax
import jax.numpy as jnp
import numpy as np
from jax.experimental import pallas as pl
from jax.experimental.pallas import tpu as pltpu

HID = 1024
NH = 16
NKV = 4
HD = 64
QM = NH // NKV
NE = 16
TOPK = 2
INTER = 1024
LIMIT = 7.0
THETA = 150000.0
CACHE = 4096
B = 8

INTERPRET = False


def _rope_cos_sin(pos):
    half = HD // 2
    i = jax.lax.broadcasted_iota(jnp.int32, (1, half), 1).astype(jnp.float32)
    inv_freq = jnp.exp(-(i / half) * np.log(THETA))
    freqs = pos * inv_freq
    return jnp.cos(freqs), jnp.sin(freqs)


def _qkv_kernel(x_ref, scale_ref, w_ref, b_ref, q_ref, k_ref, v_ref):
    x = x_ref[...]
    xs = x * jax.lax.rsqrt(jnp.mean(x * x, axis=-1, keepdims=True) + 1e-5)
    t = xs * scale_ref[...]
    qkv = jax.lax.dot_general(t, w_ref[...], (((1,), (1,)), ((), ())),
                              preferred_element_type=jnp.float32) + b_ref[...]
    q_ref[...] = qkv[:, : NH * HD]
    k = qkv[:, NH * HD:(NH + NKV) * HD]
    v = qkv[:, (NH + NKV) * HD:]
    cos, sin = _rope_cos_sin(float(CACHE))
    half = HD // 2
    pieces = []
    for h in range(NKV):
        x1 = k[:, h * HD: h * HD + half]
        x2 = k[:, h * HD + half: (h + 1) * HD]
        pieces.append(x1 * cos - x2 * sin)
        pieces.append(x2 * cos + x1 * sin)
    k_ref[...] = jnp.concatenate(pieces, axis=1)
    v_ref[...] = v


def _attn_kernel(q_ref, ck_ref, cv_ref, kn_ref, vn_ref, sink_ref,
                 attn_ref, ko_ref, vo_ref):
    sm_scale = 1.0 / np.sqrt(HD)
    half = HD // 2
    q16 = q_ref[...]
    cos, sin = _rope_cos_sin(float(CACHE))
    q1 = q16[:, :half]
    q2 = q16[:, half:]
    q16 = jnp.concatenate([q1 * cos - q2 * sin, q2 * cos + q1 * sin], axis=1)
    row = jax.lax.broadcasted_iota(jnp.int32, (NH, 1), 0) // QM
    q_exp = jnp.concatenate(
        [jnp.where(row == g, q16, 0.0) for g in range(NKV)], axis=1)

    K2 = ck_ref[0]
    V2 = cv_ref[0]
    kn = kn_ref[0]
    vn = vn_ref[0]

    scores = jax.lax.dot_general(q_exp, K2, (((1,), (1,)), ((), ())),
                                 preferred_element_type=jnp.float32) * sm_scale
    s_new = jax.lax.dot_general(q_exp, kn, (((1,), (1,)), ((), ())),
                                preferred_element_type=jnp.float32) * sm_scale
    s_sink = sink_ref[...]
    m = jnp.maximum(jnp.max(scores, axis=1, keepdims=True),
                    jnp.maximum(s_new, s_sink))
    p = jnp.exp(scores - m)
    p_new = jnp.exp(s_new - m)
    denom = (jnp.sum(p, axis=1, keepdims=True) + p_new
             + jnp.exp(s_sink - m))
    attn_all = jax.lax.dot_general(p, V2, (((1,), (0,)), ((), ())),
                                   preferred_element_type=jnp.float32)
    attn_all = (attn_all + p_new * vn) / denom
    attn_ref[...] = jnp.concatenate(
        [attn_all[g * QM:(g + 1) * QM, g * HD:(g + 1) * HD]
         for g in range(NKV)], axis=0)
    ko_ref[0, pl.ds(0, CACHE), :] = K2
    vo_ref[0, pl.ds(0, CACHE), :] = V2
    ko_ref[0, pl.ds(CACHE, 1), :] = kn
    vo_ref[0, pl.ds(CACHE, 1), :] = vn


def _post_kernel(x_ref, attn_ref, ow_ref, ob_ref, ms_ref, gw_ref, gb_ref,
                 x1_ref, t2_ref, c_ref, sched_ref):
    x1 = x_ref[...] + jax.lax.dot_general(
        attn_ref[...], ow_ref[...], (((1,), (1,)), ((), ())),
        preferred_element_type=jnp.float32) + ob_ref[...]
    x1_ref[...] = x1
    xs = x1 * jax.lax.rsqrt(jnp.mean(x1 * x1, axis=-1, keepdims=True) + 1e-5)
    t2 = xs * ms_ref[...]
    t2_ref[...] = t2
    g = jax.lax.dot_general(t2, gw_ref[...], (((1,), (1,)), ((), ())),
                            preferred_element_type=jnp.float32) + gb_ref[...]
    iota = jax.lax.broadcasted_iota(jnp.int32, (B, NE), 1)
    m1 = jnp.max(g, axis=1, keepdims=True)
    idx1 = jnp.min(jnp.where(g == m1, iota, NE), axis=1, keepdims=True)
    g2 = jnp.where(iota == idx1, -jnp.inf, g)
    m2 = jnp.max(g2, axis=1, keepdims=True)
    idx2 = jnp.min(jnp.where(g2 == m2, iota, NE), axis=1, keepdims=True)
    p1 = 1.0 / (1.0 + jnp.exp(m2 - m1))
    p2 = 1.0 - p1
    C = (jnp.where(iota == idx1, p1, 0.0)
         + jnp.where(iota == idx2, p2, 0.0))
    c_ref[...] = C
    usedf = jnp.max(jnp.where(C != 0.0, 1.0, 0.0), axis=0, keepdims=True)
    le = jax.lax.broadcasted_iota(jnp.int32, (NE, NE), 0)
    ge = jax.lax.broadcasted_iota(jnp.int32, (NE, NE), 1)
    lower = jnp.where(le <= ge, 1.0, 0.0)
    rank = jax.lax.dot_general(usedf, lower, (((1,), (0,)), ((), ())),
                               preferred_element_type=jnp.float32)
    lane = jax.lax.broadcasted_iota(jnp.int32, (NE, NE), 1)
    slot = jax.lax.broadcasted_iota(jnp.int32, (NE, NE), 0)
    cond = (usedf > 0.0) & (rank.astype(jnp.int32) == slot + 1)
    sched = jnp.max(jnp.where(cond, lane, -1), axis=1, keepdims=True)
    maxused = jnp.max(jnp.where(usedf > 0.0, lane[:1], -1), axis=1, keepdims=True)
    sched_ref[...] = jnp.where(sched < 0, maxused, sched)


def _moe_kernel(s_ref, t2_ref, x1_ref, c_ref, w1_ref, b1_ref,
                w2_ref, b2_ref, out_ref):
    i = pl.program_id(0)
    e = s_ref[i]
    t2 = t2_ref[...]
    h = jax.lax.dot_general(t2, w1_ref[0], (((1,), (1,)), ((), ())),
                            preferred_element_type=jnp.float32) + b1_ref[0]
    h3 = jnp.reshape(h, (B, INTER, 2))
    hg = h3[:, :, 0]
    hl = h3[:, :, 1]
    xg = jnp.minimum(hg, LIMIT)
    xl = jnp.clip(hl, -LIMIT, LIMIT)
    act = xg * jax.nn.sigmoid(1.702 * xg) * (xl + 1.0)
    o = jax.lax.dot_general(act, w2_ref[0], (((1,), (1,)), ((), ())),
                            preferred_element_type=jnp.float32) + b2_ref[0]
    lane = jax.lax.broadcasted_iota(jnp.int32, (B, NE), 1)
    C = c_ref[...]
    ce = jnp.sum(jnp.where(lane == e, C, 0.0), axis=1, keepdims=True)
    n_used = jnp.sum(jnp.max(jnp.where(C != 0.0, 1.0, 0.0), axis=0))
    ce = jnp.where(i.astype(jnp.float32) < n_used, ce, 0.0)

    @pl.when(i == 0)
    def _():
        out_ref[...] = x1_ref[...]

    out_ref[...] += ce * o


@jax.jit
def kernel(x, cache_k, cache_v, sinks, attn_norm_scale, qkv_w, qkv_b,
           out_w, out_b, mlp_norm_scale, gate_w, gate_b,
           mlp1_w, mlp1_b, mlp2_w, mlp2_b):
    Bq, Tq, _ = x.shape
    qkv_dim = HD * (NH + 2 * NKV)
    x2 = x.reshape(Bq, HID)

    q, k_new, v_new = pl.pallas_call(
        _qkv_kernel,
        out_shape=[
            jax.ShapeDtypeStruct((Bq, NH * HD), jnp.float32),
            jax.ShapeDtypeStruct((Bq, NKV * HD), jnp.float32),
            jax.ShapeDtypeStruct((Bq, NKV * HD), jnp.float32),
        ],
        interpret=INTERPRET,
    )(x2, attn_norm_scale.reshape(1, HID), qkv_w, qkv_b.reshape(1, qkv_dim))

    q128 = q.reshape(Bq * NH, HD)
    ck = cache_k.reshape(Bq, CACHE, NKV * HD)
    cv = cache_v.reshape(Bq, CACHE, NKV * HD)

    attn, K_out, V_out = pl.pallas_call(
        _attn_kernel,
        grid=(Bq,),
        in_specs=[
            pl.BlockSpec((NH, HD), lambda b: (b, 0)),
            pl.BlockSpec((1, CACHE, NKV * HD), lambda b: (b, 0, 0)),
            pl.BlockSpec((1, CACHE, NKV * HD), lambda b: (b, 0, 0)),
            pl.BlockSpec((1, 1, NKV * HD), lambda b: (b, 0, 0)),
            pl.BlockSpec((1, 1, NKV * HD), lambda b: (b, 0, 0)),
            pl.BlockSpec((NH, 1), lambda b: (0, 0)),
        ],
        out_specs=[
            pl.BlockSpec((NH, HD), lambda b: (b, 0)),
            pl.BlockSpec((1, CACHE + 1, NKV * HD), lambda b: (b, 0, 0)),
            pl.BlockSpec((1, CACHE + 1, NKV * HD), lambda b: (b, 0, 0)),
        ],
        out_shape=[
            jax.ShapeDtypeStruct((Bq * NH, HD), jnp.float32),
            jax.ShapeDtypeStruct((Bq, CACHE + 1, NKV * HD), jnp.float32),
            jax.ShapeDtypeStruct((Bq, CACHE + 1, NKV * HD), jnp.float32),
        ],
        interpret=INTERPRET,
    )(q128, ck, cv, k_new.reshape(Bq, 1, NKV * HD), v_new.reshape(Bq, 1, NKV * HD),
      sinks.reshape(NH, 1))

    x1, t2, C, sched = pl.pallas_call(
        _post_kernel,
        out_shape=[
            jax.ShapeDtypeStruct((Bq, HID), jnp.float32),
            jax.ShapeDtypeStruct((Bq, HID), jnp.float32),
            jax.ShapeDtypeStruct((Bq, NE), jnp.float32),
            jax.ShapeDtypeStruct((NE, 1), jnp.int32),
        ],
        interpret=INTERPRET,
    )(x2, attn.reshape(Bq, NH * HD), out_w, out_b.reshape(1, HID),
      mlp_norm_scale.reshape(1, HID), gate_w, gate_b.reshape(1, NE))

    out = pl.pallas_call(
        _moe_kernel,
        grid_spec=pltpu.PrefetchScalarGridSpec(
            num_scalar_prefetch=1,
            grid=(NE,),
            in_specs=[
                pl.BlockSpec((Bq, HID), lambda i, s: (0, 0)),
                pl.BlockSpec((Bq, HID), lambda i, s: (0, 0)),
                pl.BlockSpec((Bq, NE), lambda i, s: (0, 0)),
                pl.BlockSpec((1, 2 * INTER, HID), lambda i, s: (s[i], 0, 0)),
                pl.BlockSpec((1, 1, 2 * INTER), lambda i, s: (s[i], 0, 0)),
                pl.BlockSpec((1, HID, INTER), lambda i, s: (s[i], 0, 0)),
                pl.BlockSpec((1, 1, HID), lambda i, s: (s[i], 0, 0)),
            ],
            out_specs=pl.BlockSpec((Bq, HID), lambda i, s: (0, 0)),
        ),
        out_shape=jax.ShapeDtypeStruct((Bq, HID), jnp.float32),
        interpret=INTERPRET,
    )(sched.reshape(NE), t2, x1, C, mlp1_w, mlp1_b.reshape(NE, 1, 2 * INTER),
      mlp2_w, mlp2_b.reshape(NE, 1, HID))

    return (out.reshape(Bq, Tq, HID),
            K_out.reshape(Bq, CACHE + 1, NKV, HD),
            V_out.reshape(Bq, CACHE + 1, NKV, HD))


# speedup vs baseline: 1.0961x; 1.0226x over previous
import functools

import jax
import jax.numpy as jnp
import numpy as np
from jax.experimental import pallas as pl
from jax.experimental.pallas import tpu as pltpu

HID = 1024
NH = 16
NKV = 4
HD = 64
QM = NH // NKV
NE = 16
TOPK = 2
INTER = 1024
LIMIT = 7.0
THETA = 150000.0
CACHE = 4096
B = 8

INTERPRET = False


def _rope_cos_sin(pos):
    half = HD // 2
    i = jax.lax.broadcasted_iota(jnp.int32, (1, half), 1).astype(jnp.float32)
    inv_freq = jnp.exp(-(i / half) * np.log(THETA))
    freqs = pos * inv_freq
    return jnp.cos(freqs), jnp.sin(freqs)


def _attn_kernel(x_ref, scale_ref, w_ref, b_ref, ck_ref, cv_ref, sink_ref,
                 attn_ref, ko_ref, vo_ref, q_s, kn_s, vn_s):
    bi = pl.program_id(0)
    sm_scale = 1.0 / np.sqrt(HD)
    half = HD // 2
    cos, sin = _rope_cos_sin(float(CACHE))

    @pl.when(bi == 0)
    def _():
        x = x_ref[...]
        xs = x * jax.lax.rsqrt(jnp.mean(x * x, axis=-1, keepdims=True) + 1e-5)
        t = xs * scale_ref[...]
        qkv = jax.lax.dot_general(t, w_ref[...], (((1,), (1,)), ((), ())),
                                  preferred_element_type=jnp.float32) + b_ref[...]
        q_s[...] = qkv[:, : NH * HD]
        k = qkv[:, NH * HD:(NH + NKV) * HD]
        pieces = []
        for h in range(NKV):
            x1 = k[:, h * HD: h * HD + half]
            x2 = k[:, h * HD + half: (h + 1) * HD]
            pieces.append(x1 * cos - x2 * sin)
            pieces.append(x2 * cos + x1 * sin)
        kn_s[...] = jnp.concatenate(pieces, axis=1)
        vn_s[...] = qkv[:, (NH + NKV) * HD:]

    qrow = q_s[pl.ds(bi, 1), :]
    q16 = jnp.concatenate(
        [qrow[:, r * HD:(r + 1) * HD] for r in range(NH)], axis=0)
    q1 = q16[:, :half]
    q2 = q16[:, half:]
    q16 = jnp.concatenate([q1 * cos - q2 * sin, q2 * cos + q1 * sin], axis=1)
    row = jax.lax.broadcasted_iota(jnp.int32, (NH, 1), 0) // QM
    q_exp = jnp.concatenate(
        [jnp.where(row == g, q16, 0.0) for g in range(NKV)], axis=1)

    K2 = ck_ref[0]
    V2 = cv_ref[0]
    kn = kn_s[pl.ds(bi, 1), :]
    vn = vn_s[pl.ds(bi, 1), :]

    scores = jax.lax.dot_general(q_exp, K2, (((1,), (1,)), ((), ())),
                                 preferred_element_type=jnp.float32) * sm_scale
    s_new = jax.lax.dot_general(q_exp, kn, (((1,), (1,)), ((), ())),
                                preferred_element_type=jnp.float32) * sm_scale
    s_sink = sink_ref[...]
    m = jnp.maximum(jnp.max(scores, axis=1, keepdims=True),
                    jnp.maximum(s_new, s_sink))
    p = jnp.exp(scores - m)
    p_new = jnp.exp(s_new - m)
    denom = (jnp.sum(p, axis=1, keepdims=True) + p_new
             + jnp.exp(s_sink - m))
    attn_all = jax.lax.dot_general(p, V2, (((1,), (0,)), ((), ())),
                                   preferred_element_type=jnp.float32)
    attn_all = (attn_all + p_new * vn) / denom
    attn_ref[...] = jnp.concatenate(
        [attn_all[g * QM:(g + 1) * QM, g * HD:(g + 1) * HD]
         for g in range(NKV)], axis=0)
    ko_ref[0, pl.ds(0, CACHE), :] = K2
    vo_ref[0, pl.ds(0, CACHE), :] = V2
    ko_ref[0, pl.ds(CACHE, 1), :] = kn
    vo_ref[0, pl.ds(CACHE, 1), :] = vn


def _post_kernel(x_ref, attn_ref, ow_ref, ob_ref, ms_ref, gw_ref, gb_ref,
                 x1_ref, t2_ref, c_ref, sched_ref):
    x1 = x_ref[...] + jax.lax.dot_general(
        attn_ref[...], ow_ref[...], (((1,), (1,)), ((), ())),
        preferred_element_type=jnp.float32) + ob_ref[...]
    x1_ref[...] = x1
    xs = x1 * jax.lax.rsqrt(jnp.mean(x1 * x1, axis=-1, keepdims=True) + 1e-5)
    t2 = xs * ms_ref[...]
    t2_ref[...] = t2
    g = jax.lax.dot_general(t2, gw_ref[...], (((1,), (1,)), ((), ())),
                            preferred_element_type=jnp.float32) + gb_ref[...]
    iota = jax.lax.broadcasted_iota(jnp.int32, (B, NE), 1)
    m1 = jnp.max(g, axis=1, keepdims=True)
    idx1 = jnp.min(jnp.where(g == m1, iota, NE), axis=1, keepdims=True)
    g2 = jnp.where(iota == idx1, -jnp.inf, g)
    m2 = jnp.max(g2, axis=1, keepdims=True)
    idx2 = jnp.min(jnp.where(g2 == m2, iota, NE), axis=1, keepdims=True)
    p1 = 1.0 / (1.0 + jnp.exp(m2 - m1))
    p2 = 1.0 - p1
    C = (jnp.where(iota == idx1, p1, 0.0)
         + jnp.where(iota == idx2, p2, 0.0))
    c_ref[...] = C
    usedf = jnp.max(jnp.where(C != 0.0, 1.0, 0.0), axis=0, keepdims=True)
    le = jax.lax.broadcasted_iota(jnp.int32, (NE, NE), 0)
    ge = jax.lax.broadcasted_iota(jnp.int32, (NE, NE), 1)
    lower = jnp.where(le <= ge, 1.0, 0.0)
    rank = jax.lax.dot_general(usedf, lower, (((1,), (0,)), ((), ())),
                               preferred_element_type=jnp.float32)
    lane = jax.lax.broadcasted_iota(jnp.int32, (NE, NE), 1)
    slot = jax.lax.broadcasted_iota(jnp.int32, (NE, NE), 0)
    cond = (usedf > 0.0) & (rank.astype(jnp.int32) == slot + 1)
    sched = jnp.max(jnp.where(cond, lane, -1), axis=1, keepdims=True)
    maxused = jnp.max(jnp.where(usedf > 0.0, lane[:1], -1), axis=1, keepdims=True)
    sched_ref[...] = jnp.where(sched < 0, maxused, sched)


def _moe_kernel(s_ref, t2_ref, x1_ref, c_ref, w1_ref, b1_ref,
                w2_ref, b2_ref, out_ref):
    i = pl.program_id(0)
    e = s_ref[i]
    t2 = t2_ref[...]
    h = jax.lax.dot_general(t2, w1_ref[0], (((1,), (1,)), ((), ())),
                            preferred_element_type=jnp.float32) + b1_ref[0]
    h3 = jnp.reshape(h, (B, INTER, 2))
    hg = h3[:, :, 0]
    hl = h3[:, :, 1]
    xg = jnp.minimum(hg, LIMIT)
    xl = jnp.clip(hl, -LIMIT, LIMIT)
    act = xg * jax.nn.sigmoid(1.702 * xg) * (xl + 1.0)
    o = jax.lax.dot_general(act, w2_ref[0], (((1,), (1,)), ((), ())),
                            preferred_element_type=jnp.float32) + b2_ref[0]
    lane = jax.lax.broadcasted_iota(jnp.int32, (B, NE), 1)
    C = c_ref[...]
    ce = jnp.sum(jnp.where(lane == e, C, 0.0), axis=1, keepdims=True)
    n_used = jnp.sum(jnp.max(jnp.where(C != 0.0, 1.0, 0.0), axis=0))
    ce = jnp.where(i.astype(jnp.float32) < n_used, ce, 0.0)

    @pl.when(i == 0)
    def _():
        out_ref[...] = x1_ref[...]

    out_ref[...] += ce * o


@jax.jit
def kernel(x, cache_k, cache_v, sinks, attn_norm_scale, qkv_w, qkv_b,
           out_w, out_b, mlp_norm_scale, gate_w, gate_b,
           mlp1_w, mlp1_b, mlp2_w, mlp2_b):
    Bq, Tq, _ = x.shape
    qkv_dim = HD * (NH + 2 * NKV)
    x2 = x.reshape(Bq, HID)

    ck = cache_k.reshape(Bq, CACHE, NKV * HD)
    cv = cache_v.reshape(Bq, CACHE, NKV * HD)

    attn, K_out, V_out = pl.pallas_call(
        _attn_kernel,
        grid=(Bq,),
        in_specs=[
            pl.BlockSpec((Bq, HID), lambda b: (0, 0)),
            pl.BlockSpec((1, HID), lambda b: (0, 0)),
            pl.BlockSpec((qkv_dim, HID), lambda b: (0, 0)),
            pl.BlockSpec((1, qkv_dim), lambda b: (0, 0)),
            pl.BlockSpec((1, CACHE, NKV * HD), lambda b: (b, 0, 0)),
            pl.BlockSpec((1, CACHE, NKV * HD), lambda b: (b, 0, 0)),
            pl.BlockSpec((NH, 1), lambda b: (0, 0)),
        ],
        out_specs=[
            pl.BlockSpec((NH, HD), lambda b: (b, 0)),
            pl.BlockSpec((1, CACHE + 1, NKV * HD), lambda b: (b, 0, 0)),
            pl.BlockSpec((1, CACHE + 1, NKV * HD), lambda b: (b, 0, 0)),
        ],
        out_shape=[
            jax.ShapeDtypeStruct((Bq * NH, HD), jnp.float32),
            jax.ShapeDtypeStruct((Bq, CACHE + 1, NKV * HD), jnp.float32),
            jax.ShapeDtypeStruct((Bq, CACHE + 1, NKV * HD), jnp.float32),
        ],
        scratch_shapes=[
            pltpu.VMEM((Bq, NH * HD), jnp.float32),
            pltpu.VMEM((Bq, NKV * HD), jnp.float32),
            pltpu.VMEM((Bq, NKV * HD), jnp.float32),
        ],
        interpret=INTERPRET,
    )(x2, attn_norm_scale.reshape(1, HID), qkv_w, qkv_b.reshape(1, qkv_dim),
      ck, cv, sinks.reshape(NH, 1))

    x1, t2, C, sched = pl.pallas_call(
        _post_kernel,
        out_shape=[
            jax.ShapeDtypeStruct((Bq, HID), jnp.float32),
            jax.ShapeDtypeStruct((Bq, HID), jnp.float32),
            jax.ShapeDtypeStruct((Bq, NE), jnp.float32),
            jax.ShapeDtypeStruct((NE, 1), jnp.int32),
        ],
        interpret=INTERPRET,
    )(x2, attn.reshape(Bq, NH * HD), out_w, out_b.reshape(1, HID),
      mlp_norm_scale.reshape(1, HID), gate_w, gate_b.reshape(1, NE))

    out = pl.pallas_call(
        _moe_kernel,
        grid_spec=pltpu.PrefetchScalarGridSpec(
            num_scalar_prefetch=1,
            grid=(NE,),
            in_specs=[
                pl.BlockSpec((Bq, HID), lambda i, s: (0, 0)),
                pl.BlockSpec((Bq, HID), lambda i, s: (0, 0)),
                pl.BlockSpec((Bq, NE), lambda i, s: (0, 0)),
                pl.BlockSpec((1, 2 * INTER, HID), lambda i, s: (s[i], 0, 0)),
                pl.BlockSpec((1, 1, 2 * INTER), lambda i, s: (s[i], 0, 0)),
                pl.BlockSpec((1, HID, INTER), lambda i, s: (s[i], 0, 0)),
                pl.BlockSpec((1, 1, HID), lambda i, s: (s[i], 0, 0)),
            ],
            out_specs=pl.BlockSpec((Bq, HID), lambda i, s: (0, 0)),
        ),
        out_shape=jax.ShapeDtypeStruct((Bq, HID), jnp.float32),
        interpret=INTERPRET,
    )(sched.reshape(NE), t2, x1, C, mlp1_w, mlp1_b.reshape(NE, 1, 2 * INTER),
      mlp2_w, mlp2_b.reshape(NE, 1, HID))

    return (out.reshape(Bq, Tq, HID),
            K_out.reshape(Bq, CACHE + 1, NKV, HD),
            V_out.reshape(Bq, CACHE + 1, NKV, HD))


# speedup vs baseline: 1.1090x; 1.0118x over previous
import functools

import jax
import jax.numpy as jnp
import numpy as np
from jax.experimental import pallas as pl
from jax.experimental.pallas import tpu as pltpu

HID = 1024
NH = 16
NKV = 4
HD = 64
QM = NH // NKV
NE = 16
TOPK = 2
INTER = 1024
LIMIT = 7.0
THETA = 150000.0
CACHE = 4096
B = 8

INTERPRET = False


def _rope_cos_sin(pos):
    half = HD // 2
    i = jax.lax.broadcasted_iota(jnp.int32, (1, half), 1).astype(jnp.float32)
    inv_freq = jnp.exp(-(i / half) * np.log(THETA))
    freqs = pos * inv_freq
    return jnp.cos(freqs), jnp.sin(freqs)


def _attn_kernel(x_ref, scale_ref, w_ref, b_ref, ck_ref, cv_ref, sink_ref,
                 attn_ref, ko_ref, vo_ref, q_s, kn_s, vn_s):
    bi = pl.program_id(0)
    sm_scale = 1.0 / np.sqrt(HD)
    half = HD // 2
    cos, sin = _rope_cos_sin(float(CACHE))

    @pl.when(bi == 0)
    def _():
        x = x_ref[...]
        xs = x * jax.lax.rsqrt(jnp.mean(x * x, axis=-1, keepdims=True) + 1e-5)
        t = xs * scale_ref[...]
        qkv = jax.lax.dot_general(t, w_ref[...], (((1,), (1,)), ((), ())),
                                  preferred_element_type=jnp.float32) + b_ref[...]
        q_s[...] = qkv[:, : NH * HD]
        k = qkv[:, NH * HD:(NH + NKV) * HD]
        pieces = []
        for h in range(NKV):
            x1 = k[:, h * HD: h * HD + half]
            x2 = k[:, h * HD + half: (h + 1) * HD]
            pieces.append(x1 * cos - x2 * sin)
            pieces.append(x2 * cos + x1 * sin)
        kn_s[...] = jnp.concatenate(pieces, axis=1)
        vn_s[...] = qkv[:, (NH + NKV) * HD:]

    qrow = q_s[pl.ds(bi, 1), :]
    q16 = jnp.concatenate(
        [qrow[:, r * HD:(r + 1) * HD] for r in range(NH)], axis=0)
    q1 = q16[:, :half]
    q2 = q16[:, half:]
    q16 = jnp.concatenate([q1 * cos - q2 * sin, q2 * cos + q1 * sin], axis=1)
    row = jax.lax.broadcasted_iota(jnp.int32, (NH, 1), 0) // QM
    q_exp = jnp.concatenate(
        [jnp.where(row == g, q16, 0.0) for g in range(NKV)], axis=1)

    K2 = ck_ref[0]
    V2 = cv_ref[0]
    kn = kn_s[pl.ds(bi, 1), :]
    vn = vn_s[pl.ds(bi, 1), :]

    scores = jax.lax.dot_general(q_exp, K2, (((1,), (1,)), ((), ())),
                                 preferred_element_type=jnp.float32) * sm_scale
    s_new = jax.lax.dot_general(q_exp, kn, (((1,), (1,)), ((), ())),
                                preferred_element_type=jnp.float32) * sm_scale
    s_sink = sink_ref[...]
    m = jnp.maximum(jnp.max(scores, axis=1, keepdims=True),
                    jnp.maximum(s_new, s_sink))
    p = jnp.exp(scores - m)
    p_new = jnp.exp(s_new - m)
    denom = (jnp.sum(p, axis=1, keepdims=True) + p_new
             + jnp.exp(s_sink - m))
    attn_all = jax.lax.dot_general(p, V2, (((1,), (0,)), ((), ())),
                                   preferred_element_type=jnp.float32)
    attn_all = (attn_all + p_new * vn) / denom
    attn_ref[...] = jnp.concatenate(
        [attn_all[g * QM:(g + 1) * QM, g * HD:(g + 1) * HD]
         for g in range(NKV)], axis=0)
    ko_ref[0, pl.ds(0, CACHE), :] = K2
    vo_ref[0, pl.ds(0, CACHE), :] = V2
    ko_ref[0, pl.ds(CACHE, 1), :] = kn
    vo_ref[0, pl.ds(CACHE, 1), :] = vn


def _moe_kernel(x_ref, attn_ref, ow_ref, ob_ref, ms_ref, gw_ref, gb_ref,
                w1_ref, b1_ref, w2_ref, b2_ref, out_ref,
                x1_s, t2_s, c_s):
    e = pl.program_id(0)

    @pl.when(e == 0)
    def _():
        x1 = x_ref[...] + jax.lax.dot_general(
            attn_ref[...], ow_ref[...], (((1,), (1,)), ((), ())),
            preferred_element_type=jnp.float32) + ob_ref[...]
        x1_s[...] = x1
        xs = x1 * jax.lax.rsqrt(
            jnp.mean(x1 * x1, axis=-1, keepdims=True) + 1e-5)
        t2 = xs * ms_ref[...]
        t2_s[...] = t2
        g = jax.lax.dot_general(t2, gw_ref[...], (((1,), (1,)), ((), ())),
                                preferred_element_type=jnp.float32) + gb_ref[...]
        iota = jax.lax.broadcasted_iota(jnp.int32, (B, NE), 1)
        m1 = jnp.max(g, axis=1, keepdims=True)
        idx1 = jnp.min(jnp.where(g == m1, iota, NE), axis=1, keepdims=True)
        g2 = jnp.where(iota == idx1, -jnp.inf, g)
        m2 = jnp.max(g2, axis=1, keepdims=True)
        idx2 = jnp.min(jnp.where(g2 == m2, iota, NE), axis=1, keepdims=True)
        p1 = 1.0 / (1.0 + jnp.exp(m2 - m1))
        p2 = 1.0 - p1
        c_s[...] = (jnp.where(iota == idx1, p1, 0.0)
                    + jnp.where(iota == idx2, p2, 0.0))

    t2 = t2_s[...]
    h = jax.lax.dot_general(t2, w1_ref[0], (((1,), (1,)), ((), ())),
                            preferred_element_type=jnp.float32) + b1_ref[0]
    h3 = jnp.reshape(h, (B, INTER, 2))
    hg = h3[:, :, 0]
    hl = h3[:, :, 1]
    xg = jnp.minimum(hg, LIMIT)
    xl = jnp.clip(hl, -LIMIT, LIMIT)
    act = xg * jax.nn.sigmoid(1.702 * xg) * (xl + 1.0)
    o = jax.lax.dot_general(act, w2_ref[0], (((1,), (1,)), ((), ())),
                            preferred_element_type=jnp.float32) + b2_ref[0]
    lane = jax.lax.broadcasted_iota(jnp.int32, (B, NE), 1)
    ce = jnp.sum(jnp.where(lane == e, c_s[...], 0.0), axis=1, keepdims=True)

    @pl.when(e == 0)
    def _():
        out_ref[...] = x1_s[...]

    out_ref[...] += ce * o


@jax.jit
def kernel(x, cache_k, cache_v, sinks, attn_norm_scale, qkv_w, qkv_b,
           out_w, out_b, mlp_norm_scale, gate_w, gate_b,
           mlp1_w, mlp1_b, mlp2_w, mlp2_b):
    Bq, Tq, _ = x.shape
    qkv_dim = HD * (NH + 2 * NKV)
    x2 = x.reshape(Bq, HID)

    ck = cache_k.reshape(Bq, CACHE, NKV * HD)
    cv = cache_v.reshape(Bq, CACHE, NKV * HD)

    attn, K_out, V_out = pl.pallas_call(
        _attn_kernel,
        grid=(Bq,),
        in_specs=[
            pl.BlockSpec((Bq, HID), lambda b: (0, 0)),
            pl.BlockSpec((1, HID), lambda b: (0, 0)),
            pl.BlockSpec((qkv_dim, HID), lambda b: (0, 0)),
            pl.BlockSpec((1, qkv_dim), lambda b: (0, 0)),
            pl.BlockSpec((1, CACHE, NKV * HD), lambda b: (b, 0, 0)),
            pl.BlockSpec((1, CACHE, NKV * HD), lambda b: (b, 0, 0)),
            pl.BlockSpec((NH, 1), lambda b: (0, 0)),
        ],
        out_specs=[
            pl.BlockSpec((NH, HD), lambda b: (b, 0)),
            pl.BlockSpec((1, CACHE + 1, NKV * HD), lambda b: (b, 0, 0)),
            pl.BlockSpec((1, CACHE + 1, NKV * HD), lambda b: (b, 0, 0)),
        ],
        out_shape=[
            jax.ShapeDtypeStruct((Bq * NH, HD), jnp.float32),
            jax.ShapeDtypeStruct((Bq, CACHE + 1, NKV * HD), jnp.float32),
            jax.ShapeDtypeStruct((Bq, CACHE + 1, NKV * HD), jnp.float32),
        ],
        scratch_shapes=[
            pltpu.VMEM((Bq, NH * HD), jnp.float32),
            pltpu.VMEM((Bq, NKV * HD), jnp.float32),
            pltpu.VMEM((Bq, NKV * HD), jnp.float32),
        ],
        interpret=INTERPRET,
    )(x2, attn_norm_scale.reshape(1, HID), qkv_w, qkv_b.reshape(1, qkv_dim),
      ck, cv, sinks.reshape(NH, 1))

    out = pl.pallas_call(
        _moe_kernel,
        grid=(NE,),
        in_specs=[
            pl.BlockSpec((Bq, HID), lambda e: (0, 0)),
            pl.BlockSpec((Bq, HID), lambda e: (0, 0)),
            pl.BlockSpec((HID, NH * HD), lambda e: (0, 0)),
            pl.BlockSpec((1, HID), lambda e: (0, 0)),
            pl.BlockSpec((1, HID), lambda e: (0, 0)),
            pl.BlockSpec((NE, HID), lambda e: (0, 0)),
            pl.BlockSpec((1, NE), lambda e: (0, 0)),
            pl.BlockSpec((1, 2 * INTER, HID), lambda e: (e, 0, 0)),
            pl.BlockSpec((1, 1, 2 * INTER), lambda e: (e, 0, 0)),
            pl.BlockSpec((1, HID, INTER), lambda e: (e, 0, 0)),
            pl.BlockSpec((1, 1, HID), lambda e: (e, 0, 0)),
        ],
        out_specs=pl.BlockSpec((Bq, HID), lambda e: (0, 0)),
        out_shape=jax.ShapeDtypeStruct((Bq, HID), jnp.float32),
        scratch_shapes=[
            pltpu.VMEM((Bq, HID), jnp.float32),
            pltpu.VMEM((Bq, HID), jnp.float32),
            pltpu.VMEM((Bq, NE), jnp.float32),
        ],
        interpret=INTERPRET,
    )(x2, attn.reshape(Bq, NH * HD), out_w, out_b.reshape(1, HID),
      mlp_norm_scale.reshape(1, HID), gate_w, gate_b.reshape(1, NE),
      mlp1_w, mlp1_b.reshape(NE, 1, 2 * INTER),
      mlp2_w, mlp2_b.reshape(NE, 1, HID))

    return (out.reshape(Bq, Tq, HID),
            K_out.reshape(Bq, CACHE + 1, NKV, HD),
            V_out.reshape(Bq, CACHE + 1, NKV, HD))


# speedup vs baseline: 1.1102x; 1.0011x over previous
import jax
import jax.numpy as jnp
import numpy as np
from jax.experimental import pallas as pl
from jax.experimental.pallas import tpu as pltpu

HID = 1024
NH = 16
NKV = 4
HD = 64
QM = NH // NKV
NE = 16
TOPK = 2
INTER = 1024
LIMIT = 7.0
THETA = 150000.0
CACHE = 4096
B = 8


def _rope_cos_sin(pos):
    half = HD // 2
    i = jax.lax.broadcasted_iota(jnp.int32, (1, half), 1).astype(jnp.float32)
    inv_freq = jnp.exp(-(i / half) * np.log(THETA))
    freqs = pos * inv_freq
    return jnp.cos(freqs), jnp.sin(freqs)


def _attn_kernel(x_ref, scale_ref, w_ref, b_ref, ck_ref, cv_ref, sink_ref,
                 attn_ref, ko_ref, vo_ref, q_s, kn_s, vn_s):
    bi = pl.program_id(0)
    sm_scale = 1.0 / np.sqrt(HD)
    half = HD // 2
    cos, sin = _rope_cos_sin(float(CACHE))

    @pl.when(bi == 0)
    def _():
        x = x_ref[...]
        xs = x * jax.lax.rsqrt(jnp.mean(x * x, axis=-1, keepdims=True) + 1e-5)
        t = xs * scale_ref[...]
        qkv = jax.lax.dot_general(t, w_ref[...], (((1,), (1,)), ((), ())),
                                  preferred_element_type=jnp.float32) + b_ref[...]
        q_s[...] = qkv[:, : NH * HD]
        k = qkv[:, NH * HD:(NH + NKV) * HD]
        pieces = []
        for h in range(NKV):
            x1 = k[:, h * HD: h * HD + half]
            x2 = k[:, h * HD + half: (h + 1) * HD]
            pieces.append(x1 * cos - x2 * sin)
            pieces.append(x2 * cos + x1 * sin)
        kn_s[...] = jnp.concatenate(pieces, axis=1)
        vn_s[...] = qkv[:, (NH + NKV) * HD:]

    qrow = q_s[pl.ds(bi, 1), :]
    q16 = jnp.concatenate(
        [qrow[:, r * HD:(r + 1) * HD] for r in range(NH)], axis=0)
    q1 = q16[:, :half]
    q2 = q16[:, half:]
    q16 = jnp.concatenate([q1 * cos - q2 * sin, q2 * cos + q1 * sin], axis=1)
    row = jax.lax.broadcasted_iota(jnp.int32, (NH, 1), 0) // QM
    q_exp = jnp.concatenate(
        [jnp.where(row == g, q16, 0.0) for g in range(NKV)], axis=1)

    K2 = ck_ref[0]
    V2 = cv_ref[0]
    kn = kn_s[pl.ds(bi, 1), :]
    vn = vn_s[pl.ds(bi, 1), :]

    scores = jax.lax.dot_general(q_exp, K2, (((1,), (1,)), ((), ())),
                                 preferred_element_type=jnp.float32) * sm_scale
    s_new = jax.lax.dot_general(q_exp, kn, (((1,), (1,)), ((), ())),
                                preferred_element_type=jnp.float32) * sm_scale
    s_sink = sink_ref[...]
    m = jnp.maximum(jnp.max(scores, axis=1, keepdims=True),
                    jnp.maximum(s_new, s_sink))
    p = jnp.exp(scores - m)
    p_new = jnp.exp(s_new - m)
    denom = (jnp.sum(p, axis=1, keepdims=True) + p_new
             + jnp.exp(s_sink - m))
    attn_all = jax.lax.dot_general(p, V2, (((1,), (0,)), ((), ())),
                                   preferred_element_type=jnp.float32)
    attn_all = (attn_all + p_new * vn) / denom
    attn_ref[...] = jnp.concatenate(
        [attn_all[g * QM:(g + 1) * QM, g * HD:(g + 1) * HD]
         for g in range(NKV)], axis=0)
    ko_ref[0, pl.ds(0, CACHE), :] = K2
    vo_ref[0, pl.ds(0, CACHE), :] = V2
    ko_ref[0, pl.ds(CACHE, 1), :] = kn
    vo_ref[0, pl.ds(CACHE, 1), :] = vn


def _moe_kernel(x_ref, attn_ref, ow_ref, ob_ref, ms_ref, gw_ref, gb_ref,
                w1_ref, b1_ref, w2_ref, b2_ref, out_ref,
                x1_s, t2_s, c_s):
    e = pl.program_id(0)

    @pl.when(e == 0)
    def _():
        x1 = x_ref[...] + jax.lax.dot_general(
            attn_ref[...], ow_ref[...], (((1,), (1,)), ((), ())),
            preferred_element_type=jnp.float32) + ob_ref[...]
        x1_s[...] = x1
        xs = x1 * jax.lax.rsqrt(
            jnp.mean(x1 * x1, axis=-1, keepdims=True) + 1e-5)
        t2 = xs * ms_ref[...]
        t2_s[...] = t2
        g = jax.lax.dot_general(t2, gw_ref[...], (((1,), (1,)), ((), ())),
                                preferred_element_type=jnp.float32) + gb_ref[...]
        iota = jax.lax.broadcasted_iota(jnp.int32, (B, NE), 1)
        m1 = jnp.max(g, axis=1, keepdims=True)
        idx1 = jnp.min(jnp.where(g == m1, iota, NE), axis=1, keepdims=True)
        g2 = jnp.where(iota == idx1, -jnp.inf, g)
        m2 = jnp.max(g2, axis=1, keepdims=True)
        idx2 = jnp.min(jnp.where(g2 == m2, iota, NE), axis=1, keepdims=True)
        p1 = 1.0 / (1.0 + jnp.exp(m2 - m1))
        p2 = 1.0 - p1
        c_s[...] = (jnp.where(iota == idx1, p1, 0.0)
                    + jnp.where(iota == idx2, p2, 0.0))

    t2 = t2_s[...]
    h = jax.lax.dot_general(t2, w1_ref[0], (((1,), (1,)), ((), ())),
                            preferred_element_type=jnp.float32) + b1_ref[0]
    h3 = jnp.reshape(h, (B, INTER, 2))
    hg = h3[:, :, 0]
    hl = h3[:, :, 1]
    xg = jnp.minimum(hg, LIMIT)
    xl = jnp.clip(hl, -LIMIT, LIMIT)
    act = xg * jax.nn.sigmoid(1.702 * xg) * (xl + 1.0)
    o = jax.lax.dot_general(act, w2_ref[0], (((1,), (1,)), ((), ())),
                            preferred_element_type=jnp.float32) + b2_ref[0]
    lane = jax.lax.broadcasted_iota(jnp.int32, (B, NE), 1)
    ce = jnp.sum(jnp.where(lane == e, c_s[...], 0.0), axis=1, keepdims=True)

    @pl.when(e == 0)
    def _():
        out_ref[...] = x1_s[...]

    out_ref[...] += ce * o


@jax.jit
def kernel(x, cache_k, cache_v, sinks, attn_norm_scale, qkv_w, qkv_b,
           out_w, out_b, mlp_norm_scale, gate_w, gate_b,
           mlp1_w, mlp1_b, mlp2_w, mlp2_b):
    Bq, Tq, _ = x.shape
    qkv_dim = HD * (NH + 2 * NKV)
    x2 = x.reshape(Bq, HID)

    ck = cache_k.reshape(Bq, CACHE, NKV * HD)
    cv = cache_v.reshape(Bq, CACHE, NKV * HD)

    attn, K_out, V_out = pl.pallas_call(
        _attn_kernel,
        grid=(Bq,),
        in_specs=[
            pl.BlockSpec((Bq, HID), lambda b: (0, 0)),
            pl.BlockSpec((1, HID), lambda b: (0, 0)),
            pl.BlockSpec((qkv_dim, HID), lambda b: (0, 0)),
            pl.BlockSpec((1, qkv_dim), lambda b: (0, 0)),
            pl.BlockSpec((1, CACHE, NKV * HD), lambda b: (b, 0, 0)),
            pl.BlockSpec((1, CACHE, NKV * HD), lambda b: (b, 0, 0)),
            pl.BlockSpec((NH, 1), lambda b: (0, 0)),
        ],
        out_specs=[
            pl.BlockSpec((NH, HD), lambda b: (b, 0)),
            pl.BlockSpec((1, CACHE + 1, NKV * HD), lambda b: (b, 0, 0)),
            pl.BlockSpec((1, CACHE + 1, NKV * HD), lambda b: (b, 0, 0)),
        ],
        out_shape=[
            jax.ShapeDtypeStruct((Bq * NH, HD), jnp.float32),
            jax.ShapeDtypeStruct((Bq, CACHE + 1, NKV * HD), jnp.float32),
            jax.ShapeDtypeStruct((Bq, CACHE + 1, NKV * HD), jnp.float32),
        ],
        scratch_shapes=[
            pltpu.VMEM((Bq, NH * HD), jnp.float32),
            pltpu.VMEM((Bq, NKV * HD), jnp.float32),
            pltpu.VMEM((Bq, NKV * HD), jnp.float32),
        ],

    )(x2, attn_norm_scale.reshape(1, HID), qkv_w, qkv_b.reshape(1, qkv_dim),
      ck, cv, sinks.reshape(NH, 1))

    out = pl.pallas_call(
        _moe_kernel,
        grid=(NE,),
        in_specs=[
            pl.BlockSpec((Bq, HID), lambda e: (0, 0)),
            pl.BlockSpec((Bq, HID), lambda e: (0, 0)),
            pl.BlockSpec((HID, NH * HD), lambda e: (0, 0)),
            pl.BlockSpec((1, HID), lambda e: (0, 0)),
            pl.BlockSpec((1, HID), lambda e: (0, 0)),
            pl.BlockSpec((NE, HID), lambda e: (0, 0)),
            pl.BlockSpec((1, NE), lambda e: (0, 0)),
            pl.BlockSpec((1, 2 * INTER, HID), lambda e: (e, 0, 0)),
            pl.BlockSpec((1, 1, 2 * INTER), lambda e: (e, 0, 0)),
            pl.BlockSpec((1, HID, INTER), lambda e: (e, 0, 0)),
            pl.BlockSpec((1, 1, HID), lambda e: (e, 0, 0)),
        ],
        out_specs=pl.BlockSpec((Bq, HID), lambda e: (0, 0)),
        out_shape=jax.ShapeDtypeStruct((Bq, HID), jnp.float32),
        scratch_shapes=[
            pltpu.VMEM((Bq, HID), jnp.float32),
            pltpu.VMEM((Bq, HID), jnp.float32),
            pltpu.VMEM((Bq, NE), jnp.float32),
        ],

    )(x2, attn.reshape(Bq, NH * HD), out_w, out_b.reshape(1, HID),
      mlp_norm_scale.reshape(1, HID), gate_w, gate_b.reshape(1, NE),
      mlp1_w, mlp1_b.reshape(NE, 1, 2 * INTER),
      mlp2_w, mlp2_b.reshape(NE, 1, HID))

    return (out.reshape(Bq, Tq, HID),
            K_out.reshape(Bq, CACHE + 1, NKV, HD),
            V_out.reshape(Bq, CACHE + 1, NKV, HD))
